# Initial kernel scaffold; baseline (speedup 1.0000x reference)
import jax
import jax.numpy as jnp
from jax.experimental import pallas as pl


def kernel(Za, Qa, Q, batch_seg):
    raise NotImplementedError("write your pallas kernel here")


# speedup vs baseline: 241.8450x; 241.8450x over previous
import functools

import jax
import jax.numpy as jnp
from jax import lax
from jax.experimental import pallas as pl
from jax.experimental.pallas import tpu as pltpu
from jax.experimental.pallas import tpu_sc as plsc

L = 16
NC = 2
NS = 16
NW = NC * NS

_CP = pltpu.CompilerParams(needs_layout_passes=False)


def _wid():
    return lax.axis_index("c") * NS + lax.axis_index("s")


def _make_pass1(N, B, T, C, K):
    mesh = plsc.VectorSubcoreMesh(core_axis_name="c", subcore_axis_name="s")
    V = C // L

    @functools.partial(
        pl.kernel,
        out_type=(
            jax.ShapeDtypeStruct((NW * B,), jnp.float32),
            jax.ShapeDtypeStruct((NW * B,), jnp.float32),
        ),
        mesh=mesh,
        compiler_params=_CP,
        scratch_types=[
            pltpu.VMEM((C,), jnp.float32),
            pltpu.VMEM((C,), jnp.float32),
            pltpu.VMEM((C,), jnp.int32),
            pltpu.VMEM((C,), jnp.int32),
            pltpu.VMEM((B,), jnp.float32),
            pltpu.VMEM((B,), jnp.float32),
            pltpu.SemaphoreType.DMA,
            pltpu.SemaphoreType.DMA,
        ],
    )
    def pass1(qa_hbm, seg_hbm, psum_hbm, pcnt_hbm,
              qa_b0, qa_b1, seg_b0, seg_b1, acc_s, acc_c, sem0, sem1):
        wid = _wid()
        base = wid * T
        qa_bufs = (qa_b0, qa_b1)
        seg_bufs = (seg_b0, seg_b1)
        sems = (sem0, sem1)

        zz = jnp.zeros((L,), jnp.float32)

        def zbody(j, carry):
            acc_s[pl.ds(j * L, L)] = zz
            acc_c[pl.ds(j * L, L)] = zz
            return carry

        lax.fori_loop(0, B // L, zbody, 0)

        def start(k):
            b = k % 2
            return (
                pltpu.async_copy(qa_hbm.at[pl.ds(base + k * C, C)],
                                 qa_bufs[b], sems[b]),
                pltpu.async_copy(seg_hbm.at[pl.ds(base + k * C, C)],
                                 seg_bufs[b], sems[b]),
            )

        descs = [None] * K
        descs[0] = start(0)
        if K > 1:
            descs[1] = start(1)
        descs[0][0].wait()
        descs[0][1].wait()

        cur0 = seg_b0[pl.ds(0, L)]
        carry = (jnp.zeros((L,), jnp.float32),
                 jnp.zeros((L,), jnp.float32),
                 cur0)

        for k in range(K):
            b = k % 2
            if k > 0:
                descs[k][0].wait()
                descs[k][1].wait()
            qa_r = qa_bufs[b]
            seg_r = seg_bufs[b]

            def step(i, c, qa_r=qa_r, seg_r=seg_r):
                run_s, run_c, cur = c
                sl = pl.ds(i * L, L)
                qa = qa_r[sl]
                seg = seg_r[sl]
                changed = seg != cur
                plsc.addupdate_scatter(acc_s, [cur], run_s, mask=changed)
                plsc.addupdate_scatter(acc_c, [cur], run_c, mask=changed)
                run_s = jnp.where(changed, qa, run_s + qa)
                run_c = jnp.where(changed, jnp.full((L,), 1.0, jnp.float32),
                                  run_c + 1.0)
                return run_s, run_c, seg

            carry = lax.fori_loop(0, V, step, carry)
            if k + 2 < K:
                descs[k + 2] = start(k + 2)

        run_s, run_c, cur = carry
        plsc.addupdate_scatter(acc_s, [cur], run_s)
        plsc.addupdate_scatter(acc_c, [cur], run_c)

        pltpu.sync_copy(acc_s, psum_hbm.at[pl.ds(wid * B, B)])
        pltpu.sync_copy(acc_c, pcnt_hbm.at[pl.ds(wid * B, B)])

    return pass1


def _make_pass2(N, B, T, C, K):
    mesh = plsc.VectorSubcoreMesh(core_axis_name="c", subcore_axis_name="s")
    V = C // L

    @functools.partial(
        pl.kernel,
        out_type=(
            jax.ShapeDtypeStruct((N,), jnp.float32),
            jax.ShapeDtypeStruct((B,), jnp.float32),
        ),
        mesh=mesh,
        compiler_params=_CP,
        scratch_types=[
            pltpu.VMEM((C,), jnp.float32),
            pltpu.VMEM((C,), jnp.float32),
            pltpu.VMEM((C,), jnp.int32),
            pltpu.VMEM((C,), jnp.int32),
            pltpu.VMEM((C,), jnp.float32),
            pltpu.VMEM((C,), jnp.float32),
            pltpu.VMEM((NW * B,), jnp.float32),
            pltpu.VMEM((NW * B,), jnp.float32),
            pltpu.VMEM((B,), jnp.float32),
            pltpu.VMEM((B,), jnp.float32),
            pltpu.VMEM((B,), jnp.float32),
            pltpu.SemaphoreType.DMA,
            pltpu.SemaphoreType.DMA,
            pltpu.SemaphoreType.DMA,
            pltpu.SemaphoreType.DMA,
            pltpu.SemaphoreType.DMA,
        ],
    )
    def pass2(qa_hbm, seg_hbm, q_hbm, psum_hbm, pcnt_hbm,
              out_hbm, rawq_hbm,
              qa_b0, qa_b1, seg_b0, seg_b1, out_b0, out_b1,
              ps, pc, qv, corr, raw,
              semi0, semi1, semp, semo0, semo1):
        wid = _wid()
        base = wid * T
        qa_bufs = (qa_b0, qa_b1)
        seg_bufs = (seg_b0, seg_b1)
        out_bufs = (out_b0, out_b1)
        isems = (semi0, semi1)
        osems = (semo0, semo1)

        def start_in(k):
            b = k % 2
            return (
                pltpu.async_copy(qa_hbm.at[pl.ds(base + k * C, C)],
                                 qa_bufs[b], isems[b]),
                pltpu.async_copy(seg_hbm.at[pl.ds(base + k * C, C)],
                                 seg_bufs[b], isems[b]),
            )

        in_descs = [None] * K
        in_descs[0] = start_in(0)
        if K > 1:
            in_descs[1] = start_in(1)

        d1 = pltpu.async_copy(psum_hbm, ps, semp)
        d2 = pltpu.async_copy(pcnt_hbm, pc, semp)
        d3 = pltpu.async_copy(q_hbm, qv, semp)
        d1.wait()
        d2.wait()
        d3.wait()

        def comb(j, carry):
            s = jnp.zeros((L,), jnp.float32)
            n = jnp.zeros((L,), jnp.float32)
            for t in range(NW):
                s = s + ps[pl.ds(j * L + t * B, L)]
                n = n + pc[pl.ds(j * L + t * B, L)]
            sl = pl.ds(j * L, L)
            corr[sl] = (qv[sl] - s) / n
            raw[sl] = s
            return carry

        lax.fori_loop(0, B // L, comb, 0)

        out_descs = [None] * K
        for k in range(K):
            b = k % 2
            in_descs[k][0].wait()
            in_descs[k][1].wait()
            if k >= 2:
                out_descs[k - 2].wait()
            qa_r = qa_bufs[b]
            seg_r = seg_bufs[b]
            ob = out_bufs[b]

            def step(i, carry, qa_r=qa_r, seg_r=seg_r, ob=ob):
                sl = pl.ds(i * L, L)
                seg = seg_r[sl]
                qa = qa_r[sl]
                c = plsc.load_gather(corr, [seg])
                ob[sl] = qa + c
                return carry

            lax.fori_loop(0, V, step, 0)
            out_descs[k] = pltpu.async_copy(
                ob, out_hbm.at[pl.ds(base + k * C, C)], osems[b])
            if k + 2 < K:
                in_descs[k + 2] = start_in(k + 2)

        if K >= 2:
            out_descs[K - 2].wait()
        out_descs[K - 1].wait()

        @pl.when(wid == 0)
        def _():
            pltpu.sync_copy(raw, rawq_hbm)

    return pass2


def kernel(Za, Qa, Q, batch_seg):
    del Za
    N = Qa.shape[0]
    B = Q.shape[0]
    assert N % NW == 0
    T = N // NW

    C1 = 20000
    C2 = 4000
    assert T % C1 == 0 and T % C2 == 0 and C1 % L == 0 and C2 % L == 0

    qa = Qa.astype(jnp.float32)
    seg = batch_seg.astype(jnp.int32)
    q = Q.astype(jnp.float32)

    psum, pcnt = _make_pass1(N, B, T, C1, T // C1)(qa, seg)
    out, raw_q = _make_pass2(N, B, T, C2, T // C2)(qa, seg, q, psum, pcnt)
    return (out, raw_q)


# speedup vs baseline: 391.5209x; 1.6189x over previous
import functools

import jax
import jax.numpy as jnp
from jax import lax
from jax.experimental import pallas as pl
from jax.experimental.pallas import tpu as pltpu
from jax.experimental.pallas import tpu_sc as plsc

L = 16
NC = 2
NS = 16
NW = NC * NS

_CP = pltpu.CompilerParams(needs_layout_passes=False)


def _wid():
    return lax.axis_index("c") * NS + lax.axis_index("s")


def _make_pass1(N, B, T, C, K):
    mesh = plsc.VectorSubcoreMesh(core_axis_name="c", subcore_axis_name="s")
    V = C // L

    @functools.partial(
        pl.kernel,
        out_type=(
            jax.ShapeDtypeStruct((NW * B,), jnp.float32),
            jax.ShapeDtypeStruct((NW * B,), jnp.float32),
        ),
        mesh=mesh,
        compiler_params=_CP,
        scratch_types=[
            pltpu.VMEM((C,), jnp.float32),
            pltpu.VMEM((C,), jnp.float32),
            pltpu.VMEM((C,), jnp.int32),
            pltpu.VMEM((C,), jnp.int32),
            pltpu.VMEM((B,), jnp.float32),
            pltpu.VMEM((B,), jnp.float32),
            pltpu.SemaphoreType.DMA,
            pltpu.SemaphoreType.DMA,
        ],
    )
    def pass1(qa_hbm, seg_hbm, psum_hbm, pcnt_hbm,
              qa_b0, qa_b1, seg_b0, seg_b1, acc_s, acc_c, sem0, sem1):
        wid = _wid()
        base = wid * T
        qa_bufs = (qa_b0, qa_b1)
        seg_bufs = (seg_b0, seg_b1)
        sems = (sem0, sem1)

        zz = jnp.zeros((L,), jnp.float32)

        @plsc.parallel_loop(0, B // L, unroll=4)
        def _zero(j):
            acc_s[pl.ds(j * L, L)] = zz
            acc_c[pl.ds(j * L, L)] = zz

        def start(k):
            b = k % 2
            return (
                pltpu.async_copy(qa_hbm.at[pl.ds(base + k * C, C)],
                                 qa_bufs[b], sems[b]),
                pltpu.async_copy(seg_hbm.at[pl.ds(base + k * C, C)],
                                 seg_bufs[b], sems[b]),
            )

        descs = [None] * K
        descs[0] = start(0)
        if K > 1:
            descs[1] = start(1)
        descs[0][0].wait()
        descs[0][1].wait()

        cur0 = seg_b0[pl.ds(0, L)]
        carry = (jnp.zeros((L,), jnp.float32),
                 jnp.zeros((L,), jnp.float32),
                 cur0)

        for k in range(K):
            b = k % 2
            if k > 0:
                descs[k][0].wait()
                descs[k][1].wait()
            qa_r = qa_bufs[b]
            seg_r = seg_bufs[b]

            def step(i, c, qa_r=qa_r, seg_r=seg_r):
                run_s, run_c, cur = c
                sl = pl.ds(i * L, L)
                qa = qa_r[sl]
                seg = seg_r[sl]
                changed = seg != cur
                plsc.addupdate_scatter(acc_s, [cur], run_s, mask=changed)
                plsc.addupdate_scatter(acc_c, [cur], run_c, mask=changed)
                run_s = jnp.where(changed, qa, run_s + qa)
                run_c = jnp.where(changed, jnp.full((L,), 1.0, jnp.float32),
                                  run_c + 1.0)
                return run_s, run_c, seg

            carry = plsc.parallel_loop(0, V, unroll=8, carry=carry)(step)
            if k + 2 < K:
                descs[k + 2] = start(k + 2)

        run_s, run_c, cur = carry
        plsc.addupdate_scatter(acc_s, [cur], run_s)
        plsc.addupdate_scatter(acc_c, [cur], run_c)

        pltpu.sync_copy(acc_s, psum_hbm.at[pl.ds(wid * B, B)])
        pltpu.sync_copy(acc_c, pcnt_hbm.at[pl.ds(wid * B, B)])

    return pass1


def _make_pass2(N, B, T, C, K):
    mesh = plsc.VectorSubcoreMesh(core_axis_name="c", subcore_axis_name="s")
    V = C // L

    @functools.partial(
        pl.kernel,
        out_type=(
            jax.ShapeDtypeStruct((N,), jnp.float32),
            jax.ShapeDtypeStruct((B,), jnp.float32),
        ),
        mesh=mesh,
        compiler_params=_CP,
        scratch_types=[
            pltpu.VMEM((C,), jnp.float32),
            pltpu.VMEM((C,), jnp.float32),
            pltpu.VMEM((C,), jnp.int32),
            pltpu.VMEM((C,), jnp.int32),
            pltpu.VMEM((C,), jnp.float32),
            pltpu.VMEM((C,), jnp.float32),
            pltpu.VMEM((NW * B,), jnp.float32),
            pltpu.VMEM((NW * B,), jnp.float32),
            pltpu.VMEM((B,), jnp.float32),
            pltpu.VMEM((B,), jnp.float32),
            pltpu.VMEM((B,), jnp.float32),
            pltpu.SemaphoreType.DMA,
            pltpu.SemaphoreType.DMA,
            pltpu.SemaphoreType.DMA,
            pltpu.SemaphoreType.DMA,
            pltpu.SemaphoreType.DMA,
        ],
    )
    def pass2(qa_hbm, seg_hbm, q_hbm, psum_hbm, pcnt_hbm,
              out_hbm, rawq_hbm,
              qa_b0, qa_b1, seg_b0, seg_b1, out_b0, out_b1,
              ps, pc, qv, corr, raw,
              semi0, semi1, semp, semo0, semo1):
        wid = _wid()
        base = wid * T
        qa_bufs = (qa_b0, qa_b1)
        seg_bufs = (seg_b0, seg_b1)
        out_bufs = (out_b0, out_b1)
        isems = (semi0, semi1)
        osems = (semo0, semo1)

        def start_in(k):
            b = k % 2
            return (
                pltpu.async_copy(qa_hbm.at[pl.ds(base + k * C, C)],
                                 qa_bufs[b], isems[b]),
                pltpu.async_copy(seg_hbm.at[pl.ds(base + k * C, C)],
                                 seg_bufs[b], isems[b]),
            )

        in_descs = [None] * K
        in_descs[0] = start_in(0)
        if K > 1:
            in_descs[1] = start_in(1)

        d1 = pltpu.async_copy(psum_hbm, ps, semp)
        d2 = pltpu.async_copy(pcnt_hbm, pc, semp)
        d3 = pltpu.async_copy(q_hbm, qv, semp)
        d1.wait()
        d2.wait()
        d3.wait()

        @plsc.parallel_loop(0, B // L, unroll=2)
        def _comb(j):
            s = jnp.zeros((L,), jnp.float32)
            n = jnp.zeros((L,), jnp.float32)
            for t in range(NW):
                s = s + ps[pl.ds(j * L + t * B, L)]
                n = n + pc[pl.ds(j * L + t * B, L)]
            sl = pl.ds(j * L, L)
            corr[sl] = (qv[sl] - s) / n
            raw[sl] = s

        out_descs = [None] * K
        for k in range(K):
            b = k % 2
            in_descs[k][0].wait()
            in_descs[k][1].wait()
            if k >= 2:
                out_descs[k - 2].wait()
            qa_r = qa_bufs[b]
            seg_r = seg_bufs[b]
            ob = out_bufs[b]

            @plsc.parallel_loop(0, V, unroll=8)
            def _apply(i, qa_r=qa_r, seg_r=seg_r, ob=ob):
                sl = pl.ds(i * L, L)
                seg = seg_r[sl]
                qa = qa_r[sl]
                c = plsc.load_gather(corr, [seg])
                ob[sl] = qa + c
            out_descs[k] = pltpu.async_copy(
                ob, out_hbm.at[pl.ds(base + k * C, C)], osems[b])
            if k + 2 < K:
                in_descs[k + 2] = start_in(k + 2)

        if K >= 2:
            out_descs[K - 2].wait()
        out_descs[K - 1].wait()

        @pl.when(wid == 0)
        def _():
            pltpu.sync_copy(raw, rawq_hbm)

    return pass2


def kernel(Za, Qa, Q, batch_seg):
    del Za
    N = Qa.shape[0]
    B = Q.shape[0]
    assert N % NW == 0
    T = N // NW

    C1 = 20000
    C2 = 4000
    assert T % C1 == 0 and T % C2 == 0 and C1 % L == 0 and C2 % L == 0

    qa = Qa.astype(jnp.float32)
    seg = batch_seg.astype(jnp.int32)
    q = Q.astype(jnp.float32)

    psum, pcnt = _make_pass1(N, B, T, C1, T // C1)(qa, seg)
    out, raw_q = _make_pass2(N, B, T, C2, T // C2)(qa, seg, q, psum, pcnt)
    return (out, raw_q)


# speedup vs baseline: 474.5712x; 1.2121x over previous
import functools

import jax
import jax.numpy as jnp
from jax import lax
from jax.experimental import pallas as pl
from jax.experimental.pallas import tpu as pltpu
from jax.experimental.pallas import tpu_sc as plsc

L = 16
NC = 2
NS = 16
NW = NC * NS

_CP = pltpu.CompilerParams(needs_layout_passes=False)


def _make_pass1(N, B, T, C, K, NBUF):
    mesh = plsc.VectorSubcoreMesh(core_axis_name="c", subcore_axis_name="s")
    V = C // L

    @functools.partial(
        pl.kernel,
        out_type=(
            jax.ShapeDtypeStruct((NC * B,), jnp.float32),
            jax.ShapeDtypeStruct((NC * B,), jnp.float32),
        ),
        mesh=mesh,
        compiler_params=_CP,
        scratch_types=[
            *[pltpu.VMEM((C,), jnp.float32) for _ in range(NBUF)],
            *[pltpu.VMEM((C,), jnp.int32) for _ in range(NBUF)],
            pltpu.VMEM((B,), jnp.float32),
            pltpu.VMEM((B,), jnp.float32),
            pltpu.VMEM((B,), jnp.int32),
            pltpu.VMEM_SHARED((B,), jnp.float32),
            pltpu.VMEM_SHARED((B,), jnp.float32),
            *[pltpu.SemaphoreType.DMA for _ in range(NBUF)],
        ],
    )
    def pass1(qa_hbm, seg_hbm, psum_hbm, pcnt_hbm, *refs):
        qa_bufs = refs[0:NBUF]
        seg_bufs = refs[NBUF:2 * NBUF]
        acc_s, acc_c, idx, sh_s, sh_c = refs[2 * NBUF:2 * NBUF + 5]
        sems = refs[2 * NBUF + 5:]

        cid = lax.axis_index("c")
        sid = lax.axis_index("s")
        wid = cid * NS + sid
        base = wid * T

        zz = jnp.zeros((L,), jnp.float32)
        lane = lax.iota(jnp.int32, L)

        @plsc.parallel_loop(0, B // L, unroll=4)
        def _zero(j):
            acc_s[pl.ds(j * L, L)] = zz
            acc_c[pl.ds(j * L, L)] = zz
            idx[pl.ds(j * L, L)] = lane + j * L

        @pl.when(sid == 0)
        def _():
            pltpu.sync_copy(acc_s, sh_s)
            pltpu.sync_copy(acc_c, sh_c)

        def start(k):
            b = k % NBUF
            return (
                pltpu.async_copy(qa_hbm.at[pl.ds(base + k * C, C)],
                                 qa_bufs[b], sems[b]),
                pltpu.async_copy(seg_hbm.at[pl.ds(base + k * C, C)],
                                 seg_bufs[b], sems[b]),
            )

        descs = [None] * K
        for k in range(min(NBUF, K)):
            descs[k] = start(k)
        descs[0][0].wait()
        descs[0][1].wait()

        cur0 = seg_bufs[0][pl.ds(0, L)]
        carry = (jnp.zeros((L,), jnp.float32),
                 jnp.zeros((L,), jnp.float32),
                 cur0)

        for k in range(K):
            b = k % NBUF
            if k > 0:
                descs[k][0].wait()
                descs[k][1].wait()
            qa_r = qa_bufs[b]
            seg_r = seg_bufs[b]

            def step(i, c, qa_r=qa_r, seg_r=seg_r):
                run_s, run_c, cur = c
                sl = pl.ds(i * L, L)
                qa = qa_r[sl]
                seg = seg_r[sl]
                changed = seg != cur
                plsc.addupdate_scatter(acc_s, [cur], run_s, mask=changed)
                plsc.addupdate_scatter(acc_c, [cur], run_c, mask=changed)
                run_s = jnp.where(changed, qa, run_s + qa)
                run_c = jnp.where(changed, jnp.full((L,), 1.0, jnp.float32),
                                  run_c + 1.0)
                return run_s, run_c, seg

            carry = plsc.parallel_loop(0, V, unroll=8, carry=carry)(step)
            if k + NBUF < K:
                descs[k + NBUF] = start(k + NBUF)

        run_s, run_c, cur = carry
        plsc.addupdate_scatter(acc_s, [cur], run_s)
        plsc.addupdate_scatter(acc_c, [cur], run_c)

        plsc.subcore_barrier()
        pltpu.sync_copy(acc_s, sh_s.at[idx], add=True)
        pltpu.sync_copy(acc_c, sh_c.at[idx], add=True)
        plsc.subcore_barrier()

        @pl.when(sid == 0)
        def _():
            pltpu.sync_copy(sh_s, psum_hbm.at[pl.ds(cid * B, B)])
            pltpu.sync_copy(sh_c, pcnt_hbm.at[pl.ds(cid * B, B)])

    return pass1


def _make_pass2(N, B, T, C, K):
    mesh = plsc.VectorSubcoreMesh(core_axis_name="c", subcore_axis_name="s")
    V = C // L

    @functools.partial(
        pl.kernel,
        out_type=(
            jax.ShapeDtypeStruct((N,), jnp.float32),
            jax.ShapeDtypeStruct((B,), jnp.float32),
        ),
        mesh=mesh,
        compiler_params=_CP,
        scratch_types=[
            pltpu.VMEM((C,), jnp.float32),
            pltpu.VMEM((C,), jnp.float32),
            pltpu.VMEM((C,), jnp.int32),
            pltpu.VMEM((C,), jnp.int32),
            pltpu.VMEM((C,), jnp.float32),
            pltpu.VMEM((C,), jnp.float32),
            pltpu.VMEM((NC * B,), jnp.float32),
            pltpu.VMEM((NC * B,), jnp.float32),
            pltpu.VMEM((B,), jnp.float32),
            pltpu.VMEM((B,), jnp.float32),
            pltpu.VMEM((B,), jnp.float32),
            pltpu.SemaphoreType.DMA,
            pltpu.SemaphoreType.DMA,
            pltpu.SemaphoreType.DMA,
            pltpu.SemaphoreType.DMA,
            pltpu.SemaphoreType.DMA,
        ],
    )
    def pass2(qa_hbm, seg_hbm, q_hbm, psum_hbm, pcnt_hbm,
              out_hbm, rawq_hbm,
              qa_b0, qa_b1, seg_b0, seg_b1, out_b0, out_b1,
              ps, pc, qv, corr, raw,
              semi0, semi1, semp, semo0, semo1):
        cid = lax.axis_index("c")
        sid = lax.axis_index("s")
        wid = cid * NS + sid
        base = wid * T
        qa_bufs = (qa_b0, qa_b1)
        seg_bufs = (seg_b0, seg_b1)
        out_bufs = (out_b0, out_b1)
        isems = (semi0, semi1)
        osems = (semo0, semo1)

        def start_in(k):
            b = k % 2
            return (
                pltpu.async_copy(qa_hbm.at[pl.ds(base + k * C, C)],
                                 qa_bufs[b], isems[b]),
                pltpu.async_copy(seg_hbm.at[pl.ds(base + k * C, C)],
                                 seg_bufs[b], isems[b]),
            )

        in_descs = [None] * K
        in_descs[0] = start_in(0)
        if K > 1:
            in_descs[1] = start_in(1)

        d1 = pltpu.async_copy(psum_hbm, ps, semp)
        d2 = pltpu.async_copy(pcnt_hbm, pc, semp)
        d3 = pltpu.async_copy(q_hbm, qv, semp)
        d1.wait()
        d2.wait()
        d3.wait()

        @plsc.parallel_loop(0, B // L, unroll=4)
        def _comb(j):
            s = jnp.zeros((L,), jnp.float32)
            n = jnp.zeros((L,), jnp.float32)
            for t in range(NC):
                s = s + ps[pl.ds(j * L + t * B, L)]
                n = n + pc[pl.ds(j * L + t * B, L)]
            sl = pl.ds(j * L, L)
            corr[sl] = (qv[sl] - s) / n
            raw[sl] = s

        out_descs = [None] * K
        for k in range(K):
            b = k % 2
            in_descs[k][0].wait()
            in_descs[k][1].wait()
            if k >= 2:
                out_descs[k - 2].wait()
            qa_r = qa_bufs[b]
            seg_r = seg_bufs[b]
            ob = out_bufs[b]

            @plsc.parallel_loop(0, V, unroll=8)
            def _apply(i, qa_r=qa_r, seg_r=seg_r, ob=ob):
                sl = pl.ds(i * L, L)
                seg = seg_r[sl]
                qa = qa_r[sl]
                c = plsc.load_gather(corr, [seg])
                ob[sl] = qa + c

            out_descs[k] = pltpu.async_copy(
                ob, out_hbm.at[pl.ds(base + k * C, C)], osems[b])
            if k + 2 < K:
                in_descs[k + 2] = start_in(k + 2)

        if K >= 2:
            out_descs[K - 2].wait()
        out_descs[K - 1].wait()

        @pl.when(wid == 0)
        def _():
            pltpu.sync_copy(raw, rawq_hbm)

    return pass2


def kernel(Za, Qa, Q, batch_seg):
    del Za
    N = Qa.shape[0]
    B = Q.shape[0]
    assert N % NW == 0
    T = N // NW

    C1 = 10000
    C2 = 10000
    assert T % C1 == 0 and T % C2 == 0 and C1 % L == 0 and C2 % L == 0

    qa = Qa.astype(jnp.float32)
    seg = batch_seg.astype(jnp.int32)
    q = Q.astype(jnp.float32)

    psum, pcnt = _make_pass1(N, B, T, C1, T // C1, 4)(qa, seg)
    out, raw_q = _make_pass2(N, B, T, C2, T // C2)(qa, seg, q, psum, pcnt)
    return (out, raw_q)
